# baseline (speedup 1.0000x reference)
import functools

import jax
import jax.numpy as jnp
from jax import lax
from jax.experimental import pallas as pl
from jax.experimental.pallas import tpu as pltpu
from jax.experimental.pallas import tpu_sc as plsc

DIM = 128
NV = 10000
NC = 40000
E = 160000
R = 8

NVP = 10240
NCP = 40960

NCORES = 2
NSUB = 16
NW = NCORES * NSUB
E2 = 2 * E
E2_PAD = 327680
STRIPE = E2_PAD // NW
CAP = STRIPE
K = 128
MAXB = CAP // K
NBUF = 4

SLAB_V, NSLAB_V = 5120, 2
SLAB_C, NSLAB_C = 5120, 8

BLK_L = 1024
BLK_C = 2048

def _sc_mesh():
  return plsc.VectorSubcoreMesh(
      core_axis_name="c", subcore_axis_name="s",
      num_cores=NCORES, num_subcores=NSUB)


_SC_PARAMS = pltpu.CompilerParams(needs_layout_passes=False)


def _make_partition(slab, nslab):
  dump = slab

  def body(sh, dh,
           rsh, rdh, ch,
           stripe_s, stripe_d, stage_s, stage_d, cnts_v):
    c = lax.axis_index("c")
    t = lax.axis_index("s")
    w = c * NSUB + t
    iota = lax.iota(jnp.int32, 16)
    z16 = jnp.zeros((16,), jnp.int32)
    d16 = jnp.full((16,), dump, jnp.int32)

    pltpu.sync_copy(sh.at[pl.ds(w * STRIPE, STRIPE)], stripe_s)
    pltpu.sync_copy(dh.at[pl.ds(w * STRIPE, STRIPE)], stripe_d)
    cnts = z16
    for s in range(nslab):
      def prefill(i, _):
        stage_s[pl.ds(i * 16, 16)] = z16
        stage_d[pl.ds(i * 16, 16)] = d16
        return 0
      lax.fori_loop(0, CAP // 16, prefill, 0)

      lo = s * slab

      def scan(g, cnt):
        s_ids = stripe_s[pl.ds(g * 16, 16)]
        local = stripe_d[pl.ds(g * 16, 16)] - lo
        valid = (local >= 0) & (local < slab)
        ones = jnp.where(valid, 1, 0).astype(jnp.int32)
        pos = cnt + plsc.cumsum(ones) - 1
        plsc.store_scatter(stage_s, [pos], s_ids, mask=valid)
        plsc.store_scatter(stage_d, [pos], local, mask=valid)
        return cnt + plsc.all_reduce_population_count(valid)

      cnt = lax.fori_loop(0, STRIPE // 16, scan, z16)
      nb = (cnt + (K - 1)) // K
      cnts = jnp.where(iota == s, nb, cnts)
      base = (s * NW + w) * CAP
      pltpu.sync_copy(stage_s, rsh.at[pl.ds(base, CAP)])
      pltpu.sync_copy(stage_d, rdh.at[pl.ds(base, CAP)])
    cnts_v[...] = cnts
    pltpu.sync_copy(cnts_v, ch.at[w])

  reg = jax.ShapeDtypeStruct((nslab * NW * CAP,), jnp.int32)
  cnt = jax.ShapeDtypeStruct((NW, 16), jnp.int32)
  return pl.kernel(
      body,
      out_type=(reg, reg, cnt),
      mesh=_sc_mesh(),
      compiler_params=_SC_PARAMS,
      scratch_types=[
          pltpu.VMEM((STRIPE,), jnp.int32),
          pltpu.VMEM((STRIPE,), jnp.int32),
          pltpu.VMEM((CAP,), jnp.int32),
          pltpu.VMEM((CAP,), jnp.int32),
          pltpu.VMEM((16,), jnp.int32),
      ])


def _make_segsum(slab, nslab):
  out_rows = nslab * slab
  rpt = slab // NSUB
  zr = (slab + 16) // NSUB
  nfull, rem = zr // K, zr % K

  def body(m_h, rsh, rdh, c_h,
           out_h,
           spmem, idx_s, idx_d, rows, ca, cb, gsem, ssem):
    c = lax.axis_index("c")
    t = lax.axis_index("s")
    iota = lax.iota(jnp.int32, 16)
    fz = jnp.zeros((16,), jnp.float32)

    pltpu.sync_copy(c_h.at[2 * t], ca)
    pltpu.sync_copy(c_h.at[2 * t + 1], cb)

    for sl in range(nslab // NCORES):
      s = NCORES * sl + c
      def zrow(i, _):
        for j in range(DIM // 16):
          rows[i, pl.ds(j * 16, 16)] = fz
        return 0
      lax.fori_loop(0, K, zrow, 0)
      zb = t * zr
      for kk in range(nfull):
        pltpu.sync_copy(rows.at[pl.ds(0, K)], spmem.at[pl.ds(zb + kk * K, K)])
      if rem:
        pltpu.sync_copy(rows.at[pl.ds(0, rem)],
                        spmem.at[pl.ds(zb + nfull * K, rem)])
      plsc.subcore_barrier()

      for r, cbuf in ((0, ca), (1, cb)):
        w_src = 2 * t + r
        nb = jnp.sum(jnp.where(iota == s, cbuf[...], 0))
        base = (s * NW + w_src) * CAP

        def chunk(g, _):
          gb = base + g * (NBUF * K)
          pltpu.sync_copy(rsh.at[pl.ds(gb, NBUF * K)], idx_s)
          pltpu.sync_copy(rdh.at[pl.ds(gb, NBUF * K)], idx_d)
          pltpu.async_copy(m_h.at[idx_s], rows, gsem).wait()
          pltpu.async_copy(rows, spmem.at[idx_d], ssem, add=True).wait()
          return 0
        lax.fori_loop(0, (nb + NBUF - 1) // NBUF, chunk, 0)

      plsc.subcore_barrier()
      pltpu.sync_copy(spmem.at[pl.ds(t * rpt, rpt)],
                      out_h.at[pl.ds(s * slab + t * rpt, rpt)])
      plsc.subcore_barrier()

  return pl.kernel(
      body,
      out_type=jax.ShapeDtypeStruct((out_rows, DIM), jnp.float32),
      mesh=_sc_mesh(),
      compiler_params=_SC_PARAMS,
      scratch_types=[
          pltpu.VMEM_SHARED((slab + 16, DIM), jnp.float32),
          pltpu.VMEM((NBUF * K,), jnp.int32),
          pltpu.VMEM((NBUF * K,), jnp.int32),
          pltpu.VMEM((NBUF * K, DIM), jnp.float32),
          pltpu.VMEM((16,), jnp.int32),
          pltpu.VMEM((16,), jnp.int32),
          pltpu.SemaphoreType.DMA,
          pltpu.SemaphoreType.DMA,
      ])


def _mlp3(x, p):
  w1, b1, w2, b2, w3, b3 = p
  h = jax.nn.relu(jnp.dot(x, w1, preferred_element_type=jnp.float32) + b1)
  h = jax.nn.relu(jnp.dot(h, w2, preferred_element_type=jnp.float32) + b2)
  return jnp.dot(h, w3, preferred_element_type=jnp.float32) + b3


def _msg_body(x_ref, *refs):
  s = pl.program_id(0)
  x = x_ref[...]
  out_ref = refs[12]

  @pl.when(s == 0)
  def _():
    out_ref[...] = _mlp3(x, [r[...] for r in refs[:6]])

  @pl.when(s == 1)
  def _():
    out_ref[...] = _mlp3(x, [r[...] for r in refs[6:12]])


def _make_msg(n_rows, blk):
  nblk = n_rows // blk
  full = pl.BlockSpec((DIM, DIM), lambda s, i: (0, 0))
  bias = pl.BlockSpec((1, DIM), lambda s, i: (0, 0))
  xrow = pl.BlockSpec((blk, DIM), lambda s, i: (i, 0))
  orow = pl.BlockSpec((blk, DIM), lambda s, i: (s * nblk + i, 0))
  return pl.pallas_call(
      _msg_body,
      grid=(2, nblk),
      in_specs=[xrow] + [full, bias] * 6,
      out_specs=orow,
      out_shape=jax.ShapeDtypeStruct((2 * n_rows, DIM), jnp.float32),
  )


def _lstm_body(x_ref, h_ref, c_ref, wih_ref, whh_ref, b_ref, h2_ref, c2_ref):
  g = (jnp.dot(x_ref[...], wih_ref[...], preferred_element_type=jnp.float32)
       + jnp.dot(h_ref[...], whh_ref[...], preferred_element_type=jnp.float32)
       + b_ref[...])
  i = g[:, :DIM]
  f = g[:, DIM:2 * DIM]
  gg = g[:, 2 * DIM:3 * DIM]
  o = g[:, 3 * DIM:]
  c2 = jax.nn.sigmoid(f) * c_ref[...] + jax.nn.sigmoid(i) * jnp.tanh(gg)
  h2_ref[...] = jax.nn.sigmoid(o) * jnp.tanh(c2)
  c2_ref[...] = c2


def _make_lstm(n_rows, blk):
  row = pl.BlockSpec((blk, DIM), lambda i: (i, 0))
  wfull = pl.BlockSpec((DIM, 4 * DIM), lambda i: (0, 0))
  bfull = pl.BlockSpec((1, 4 * DIM), lambda i: (0, 0))
  return pl.pallas_call(
      _lstm_body,
      grid=(n_rows // blk,),
      in_specs=[row, row, row, wfull, wfull, bfull],
      out_specs=[row, row],
      out_shape=[jax.ShapeDtypeStruct((n_rows, DIM), jnp.float32)] * 2,
  )


def _vote_body(x_ref, *refs):
  p = [r[...] for r in refs[:6]]
  refs[6][...] = _mlp3(x_ref[...], p)


def _make_vote(n_rows, blk):
  full = pl.BlockSpec((DIM, DIM), lambda i: (0, 0))
  bias = pl.BlockSpec((1, DIM), lambda i: (0, 0))
  row = pl.BlockSpec((blk, DIM), lambda i: (i, 0))
  return pl.pallas_call(
      _vote_body,
      grid=(n_rows // blk,),
      in_specs=[row] + [full, bias] * 3,
      out_specs=row,
      out_shape=jax.ShapeDtypeStruct((n_rows, DIM), jnp.float32),
  )


def _tmlp(p):
  w1, b1, w2, b2, w3, b3 = p
  return (w1.T, b1.reshape(1, DIM), w2.T, b2.reshape(1, DIM),
          w3.T, b3.reshape(1, -1))


def _merge_edges(src_p, src_n, dst_p, dst_n, src_off):
  pad = E2_PAD - E2
  src = jnp.concatenate([src_p.astype(jnp.int32),
                         src_n.astype(jnp.int32) + src_off,
                         jnp.zeros((pad,), jnp.int32)])
  dst = jnp.concatenate([dst_p.astype(jnp.int32),
                         dst_n.astype(jnp.int32),
                         jnp.full((pad,), 1 << 28, jnp.int32)])
  return src, dst


def kernel(L_init_W, L_init_b, C_init_W, C_init_b, L_msg_pos, L_msg_neg,
           C_msg_pos, C_msg_neg, L_update, C_update, var_vote,
           var_idx_pos, cls_idx_pos, var_idx_neg, cls_idx_neg):
  lmp, lmn = _tmlp(L_msg_pos), _tmlp(L_msg_neg)
  cmp_, cmn = _tmlp(C_msg_pos), _tmlp(C_msg_neg)

  def _tlstm(p):
    wih, whh, bih, bhh = p
    return wih.T, whh.T, (bih + bhh).reshape(1, 4 * DIM)
  l_wih, l_whh, l_b = _tlstm(L_update)
  c_wih, c_whh, c_b = _tlstm(C_update)

  vw1, vb1, vw2, vb2, vw3, vb3 = _tmlp(var_vote)
  vw3p = jnp.zeros((DIM, DIM), jnp.float32).at[:, :1].set(vw3)
  vb3p = jnp.zeros((1, DIM), jnp.float32).at[:, :1].set(vb3)

  c_src, c_dst = _merge_edges(var_idx_pos, var_idx_neg,
                              cls_idx_pos, cls_idx_neg, NVP)
  v_src, v_dst = _merge_edges(cls_idx_pos, cls_idx_neg,
                              var_idx_pos, var_idx_neg, NCP)

  L_h = jnp.broadcast_to((L_init_W[:, 0] + L_init_b).reshape(1, DIM),
                         (NVP, DIM))
  C_h = jnp.broadcast_to((C_init_W[:, 0] + C_init_b).reshape(1, DIM),
                         (NCP, DIM))
  L_c = jnp.zeros((NVP, DIM), jnp.float32)
  C_c = jnp.zeros((NCP, DIM), jnp.float32)

  part_c = _make_partition(SLAB_C, NSLAB_C)
  part_v = _make_partition(SLAB_V, NSLAB_V)
  c_regs = part_c(c_src, c_dst)
  v_regs = part_v(v_src, v_dst)

  seg_c = _make_segsum(SLAB_C, NSLAB_C)
  seg_v = _make_segsum(SLAB_V, NSLAB_V)

  msg_l = _make_msg(NVP, BLK_L)
  msg_c = _make_msg(NCP, BLK_C)
  lstm_l = _make_lstm(NVP, BLK_L)
  lstm_c = _make_lstm(NCP, BLK_C)
  vote = _make_vote(NVP, BLK_L)

  for _ in range(R):
    Lm = msg_l(L_h, *lmp, *lmn)
    LC = seg_c(Lm, *c_regs)
    C_h, C_c = lstm_c(LC, C_h, C_c, c_wih, c_whh, c_b)
    Cm = msg_c(C_h, *cmp_, *cmn)
    CL = seg_v(Cm, *v_regs)
    L_h, L_c = lstm_l(CL, L_h, L_c, l_wih, l_whh, l_b)

  out = vote(L_h, vw1, vb1, vw2, vb2, vw3p, vb3p)
  return out[:NV, :1]


# speedup vs baseline: 2.7181x; 2.7181x over previous
import functools

import jax
import jax.numpy as jnp
from jax import lax
from jax.experimental import pallas as pl
from jax.experimental.pallas import tpu as pltpu
from jax.experimental.pallas import tpu_sc as plsc

DIM = 128
NV = 10000
NC = 40000
E = 160000
R = 8

NVP = 10240
NCP = 40960

NCORES = 2
NSUB = 16
NW = NCORES * NSUB
E2 = 2 * E
E2_PAD = 327680
STRIPE = E2_PAD // NW
CAP = STRIPE
K = 128
MAXB = CAP // K
NBUF = 4

SLAB_V, NSLAB_V = 5120, 2
SLAB_C, NSLAB_C = 5120, 8

BLK_L = 1024
BLK_C = 2048

def _sc_mesh():
  return plsc.VectorSubcoreMesh(
      core_axis_name="c", subcore_axis_name="s",
      num_cores=NCORES, num_subcores=NSUB)


_SC_PARAMS = pltpu.CompilerParams(needs_layout_passes=False)


def _make_partition(slab, nslab):
  dump = slab

  def body(sh, dh,
           rsh, rdh, ch,
           stripe_s, stripe_d, stage_s, stage_d, cnts_v):
    c = lax.axis_index("c")
    t = lax.axis_index("s")
    w = c * NSUB + t
    iota = lax.iota(jnp.int32, 16)
    z16 = jnp.zeros((16,), jnp.int32)
    d16 = jnp.full((16,), dump, jnp.int32)

    pltpu.sync_copy(sh.at[pl.ds(w * STRIPE, STRIPE)], stripe_s)
    pltpu.sync_copy(dh.at[pl.ds(w * STRIPE, STRIPE)], stripe_d)
    cnts = z16
    for s in range(nslab):
      def prefill(i, _):
        for j in range(K // 16):
          stage_s[i, pl.ds(j * 16, 16)] = z16
          stage_d[i, pl.ds(j * 16, 16)] = d16
        return 0
      lax.fori_loop(0, MAXB, prefill, 0)

      lo = s * slab

      def scan(g, cnt):
        s_ids = stripe_s[pl.ds(g * 16, 16)]
        local = stripe_d[pl.ds(g * 16, 16)] - lo
        valid = (local >= 0) & (local < slab)
        ones = jnp.where(valid, 1, 0).astype(jnp.int32)
        pos = cnt + plsc.cumsum(ones) - 1
        row = jnp.right_shift(pos, 7)
        col = jnp.bitwise_and(pos, K - 1)
        plsc.store_scatter(stage_s, [row, col], s_ids, mask=valid)
        plsc.store_scatter(stage_d, [row, col], local, mask=valid)
        return cnt + plsc.all_reduce_population_count(valid)

      cnt = lax.fori_loop(0, STRIPE // 16, scan, z16)
      nb = (cnt + (K - 1)) // K
      cnts = jnp.where(iota == s, nb, cnts)
      base = (s * NW + w) * MAXB
      pltpu.sync_copy(stage_s, rsh.at[pl.ds(base, MAXB)])
      pltpu.sync_copy(stage_d, rdh.at[pl.ds(base, MAXB)])
    cnts_v[...] = cnts
    pltpu.sync_copy(cnts_v, ch.at[w])

  reg = jax.ShapeDtypeStruct((nslab * NW * MAXB, K), jnp.int32)
  cnt = jax.ShapeDtypeStruct((NW, 16), jnp.int32)
  return pl.kernel(
      body,
      out_type=(reg, reg, cnt),
      mesh=_sc_mesh(),
      compiler_params=_SC_PARAMS,
      scratch_types=[
          pltpu.VMEM((STRIPE,), jnp.int32),
          pltpu.VMEM((STRIPE,), jnp.int32),
          pltpu.VMEM((MAXB, K), jnp.int32),
          pltpu.VMEM((MAXB, K), jnp.int32),
          pltpu.VMEM((16,), jnp.int32),
      ])


def _make_segsum(slab, nslab):
  out_rows = nslab * slab
  rpt = slab // NSUB
  zr = (slab + 16) // NSUB
  nfull, rem = zr // K, zr % K

  def body(m_h, rsh, rdh, c_h,
           out_h,
           spmem, idx_s, idx_d, rows, ca, cb, gsem, ssem):
    c = lax.axis_index("c")
    t = lax.axis_index("s")
    iota = lax.iota(jnp.int32, 16)
    fz = jnp.zeros((16,), jnp.float32)

    pltpu.sync_copy(c_h.at[2 * t], ca)
    pltpu.sync_copy(c_h.at[2 * t + 1], cb)

    for sl in range(nslab // NCORES):
      s = NCORES * sl + c
      def zrow(i, _):
        for j in range(DIM // 16):
          rows[0, i, pl.ds(j * 16, 16)] = fz
        return 0
      lax.fori_loop(0, K, zrow, 0)
      zb = t * zr
      for kk in range(nfull):
        pltpu.sync_copy(rows.at[0], spmem.at[pl.ds(zb + kk * K, K)])
      if rem:
        pltpu.sync_copy(rows.at[0, pl.ds(0, rem)],
                        spmem.at[pl.ds(zb + nfull * K, rem)])
      plsc.subcore_barrier()

      for r, cbuf in ((0, ca), (1, cb)):
        w_src = 2 * t + r
        nb = jnp.sum(jnp.where(iota == s, cbuf[...], 0))
        base = (s * NW + w_src) * MAXB
        pltpu.sync_copy(rsh.at[pl.ds(base, MAXB)], idx_s)
        pltpu.sync_copy(rdh.at[pl.ds(base, MAXB)], idx_d)

        def group(g, _):
          gb = g * NBUF
          for j in range(NBUF):
            b = gb + j
            @pl.when(b < nb)
            def _():
              pltpu.async_copy(m_h.at[idx_s.at[b]], rows.at[j], gsem)
          for j in range(NBUF):
            b = gb + j
            @pl.when(b < nb)
            def _():
              pltpu.make_async_copy(m_h.at[idx_s.at[b]], rows.at[j],
                                    gsem).wait()
              pltpu.async_copy(rows.at[j], spmem.at[idx_d.at[b]], ssem,
                               add=True)
          for j in range(NBUF):
            b = gb + j
            @pl.when(b < nb)
            def _():
              pltpu.make_async_copy(rows.at[j], spmem.at[idx_d.at[b]],
                                    ssem).wait()
          return 0
        lax.fori_loop(0, (nb + NBUF - 1) // NBUF, group, 0)

      plsc.subcore_barrier()
      pltpu.sync_copy(spmem.at[pl.ds(t * rpt, rpt)],
                      out_h.at[pl.ds(s * slab + t * rpt, rpt)])
      plsc.subcore_barrier()

  return pl.kernel(
      body,
      out_type=jax.ShapeDtypeStruct((out_rows, DIM), jnp.float32),
      mesh=_sc_mesh(),
      compiler_params=_SC_PARAMS,
      scratch_types=[
          pltpu.VMEM_SHARED((slab + 16, DIM), jnp.float32),
          pltpu.VMEM((MAXB, K), jnp.int32),
          pltpu.VMEM((MAXB, K), jnp.int32),
          pltpu.VMEM((NBUF, K, DIM), jnp.float32),
          pltpu.VMEM((16,), jnp.int32),
          pltpu.VMEM((16,), jnp.int32),
          pltpu.SemaphoreType.DMA,
          pltpu.SemaphoreType.DMA,
      ])


def _mlp3(x, p):
  w1, b1, w2, b2, w3, b3 = p
  h = jax.nn.relu(jnp.dot(x, w1, preferred_element_type=jnp.float32) + b1)
  h = jax.nn.relu(jnp.dot(h, w2, preferred_element_type=jnp.float32) + b2)
  return jnp.dot(h, w3, preferred_element_type=jnp.float32) + b3


def _msg_body(x_ref, *refs):
  s = pl.program_id(0)
  x = x_ref[...]
  out_ref = refs[12]

  @pl.when(s == 0)
  def _():
    out_ref[...] = _mlp3(x, [r[...] for r in refs[:6]])

  @pl.when(s == 1)
  def _():
    out_ref[...] = _mlp3(x, [r[...] for r in refs[6:12]])


def _make_msg(n_rows, blk):
  nblk = n_rows // blk
  full = pl.BlockSpec((DIM, DIM), lambda s, i: (0, 0))
  bias = pl.BlockSpec((1, DIM), lambda s, i: (0, 0))
  xrow = pl.BlockSpec((blk, DIM), lambda s, i: (i, 0))
  orow = pl.BlockSpec((blk, DIM), lambda s, i: (s * nblk + i, 0))
  return pl.pallas_call(
      _msg_body,
      grid=(2, nblk),
      in_specs=[xrow] + [full, bias] * 6,
      out_specs=orow,
      out_shape=jax.ShapeDtypeStruct((2 * n_rows, DIM), jnp.float32),
  )


def _lstm_body(x_ref, h_ref, c_ref, wih_ref, whh_ref, b_ref, h2_ref, c2_ref):
  g = (jnp.dot(x_ref[...], wih_ref[...], preferred_element_type=jnp.float32)
       + jnp.dot(h_ref[...], whh_ref[...], preferred_element_type=jnp.float32)
       + b_ref[...])
  i = g[:, :DIM]
  f = g[:, DIM:2 * DIM]
  gg = g[:, 2 * DIM:3 * DIM]
  o = g[:, 3 * DIM:]
  c2 = jax.nn.sigmoid(f) * c_ref[...] + jax.nn.sigmoid(i) * jnp.tanh(gg)
  h2_ref[...] = jax.nn.sigmoid(o) * jnp.tanh(c2)
  c2_ref[...] = c2


def _make_lstm(n_rows, blk):
  row = pl.BlockSpec((blk, DIM), lambda i: (i, 0))
  wfull = pl.BlockSpec((DIM, 4 * DIM), lambda i: (0, 0))
  bfull = pl.BlockSpec((1, 4 * DIM), lambda i: (0, 0))
  return pl.pallas_call(
      _lstm_body,
      grid=(n_rows // blk,),
      in_specs=[row, row, row, wfull, wfull, bfull],
      out_specs=[row, row],
      out_shape=[jax.ShapeDtypeStruct((n_rows, DIM), jnp.float32)] * 2,
  )


def _vote_body(x_ref, *refs):
  p = [r[...] for r in refs[:6]]
  refs[6][...] = _mlp3(x_ref[...], p)


def _make_vote(n_rows, blk):
  full = pl.BlockSpec((DIM, DIM), lambda i: (0, 0))
  bias = pl.BlockSpec((1, DIM), lambda i: (0, 0))
  row = pl.BlockSpec((blk, DIM), lambda i: (i, 0))
  return pl.pallas_call(
      _vote_body,
      grid=(n_rows // blk,),
      in_specs=[row] + [full, bias] * 3,
      out_specs=row,
      out_shape=jax.ShapeDtypeStruct((n_rows, DIM), jnp.float32),
  )


def _tmlp(p):
  w1, b1, w2, b2, w3, b3 = p
  return (w1.T, b1.reshape(1, DIM), w2.T, b2.reshape(1, DIM),
          w3.T, b3.reshape(1, -1))


def _merge_edges(src_p, src_n, dst_p, dst_n, src_off):
  pad = E2_PAD - E2
  src = jnp.concatenate([src_p.astype(jnp.int32),
                         src_n.astype(jnp.int32) + src_off,
                         jnp.zeros((pad,), jnp.int32)])
  dst = jnp.concatenate([dst_p.astype(jnp.int32),
                         dst_n.astype(jnp.int32),
                         jnp.full((pad,), 1 << 28, jnp.int32)])
  return src, dst


def kernel(L_init_W, L_init_b, C_init_W, C_init_b, L_msg_pos, L_msg_neg,
           C_msg_pos, C_msg_neg, L_update, C_update, var_vote,
           var_idx_pos, cls_idx_pos, var_idx_neg, cls_idx_neg):
  lmp, lmn = _tmlp(L_msg_pos), _tmlp(L_msg_neg)
  cmp_, cmn = _tmlp(C_msg_pos), _tmlp(C_msg_neg)

  def _tlstm(p):
    wih, whh, bih, bhh = p
    return wih.T, whh.T, (bih + bhh).reshape(1, 4 * DIM)
  l_wih, l_whh, l_b = _tlstm(L_update)
  c_wih, c_whh, c_b = _tlstm(C_update)

  vw1, vb1, vw2, vb2, vw3, vb3 = _tmlp(var_vote)
  vw3p = jnp.zeros((DIM, DIM), jnp.float32).at[:, :1].set(vw3)
  vb3p = jnp.zeros((1, DIM), jnp.float32).at[:, :1].set(vb3)

  c_src, c_dst = _merge_edges(var_idx_pos, var_idx_neg,
                              cls_idx_pos, cls_idx_neg, NVP)
  v_src, v_dst = _merge_edges(cls_idx_pos, cls_idx_neg,
                              var_idx_pos, var_idx_neg, NCP)

  L_h = jnp.broadcast_to((L_init_W[:, 0] + L_init_b).reshape(1, DIM),
                         (NVP, DIM))
  C_h = jnp.broadcast_to((C_init_W[:, 0] + C_init_b).reshape(1, DIM),
                         (NCP, DIM))
  L_c = jnp.zeros((NVP, DIM), jnp.float32)
  C_c = jnp.zeros((NCP, DIM), jnp.float32)

  part_c = _make_partition(SLAB_C, NSLAB_C)
  part_v = _make_partition(SLAB_V, NSLAB_V)
  c_regs = part_c(c_src, c_dst)
  v_regs = part_v(v_src, v_dst)

  seg_c = _make_segsum(SLAB_C, NSLAB_C)
  seg_v = _make_segsum(SLAB_V, NSLAB_V)

  msg_l = _make_msg(NVP, BLK_L)
  msg_c = _make_msg(NCP, BLK_C)
  lstm_l = _make_lstm(NVP, BLK_L)
  lstm_c = _make_lstm(NCP, BLK_C)
  vote = _make_vote(NVP, BLK_L)

  for _ in range(R):
    Lm = msg_l(L_h, *lmp, *lmn)
    LC = seg_c(Lm, *c_regs)
    C_h, C_c = lstm_c(LC, C_h, C_c, c_wih, c_whh, c_b)
    Cm = msg_c(C_h, *cmp_, *cmn)
    CL = seg_v(Cm, *v_regs)
    L_h, L_c = lstm_l(CL, L_h, L_c, l_wih, l_whh, l_b)

  out = vote(L_h, vw1, vb1, vw2, vb2, vw3p, vb3p)
  return out[:NV, :1]
